# Initial kernel scaffold; baseline (speedup 1.0000x reference)
import functools

import jax
import jax.numpy as jnp
from jax import lax
from jax.experimental import pallas as pl
from jax.experimental.pallas import tpu as pltpu
from jax.experimental.pallas import tpu_sc as plsc

NC = 2
NS = 16
NW = NC * NS

SUB = 128
K = 10
CHUNK = SUB * K


def _gather_call(n_rows, vocab, d):
    n_per_w = n_rows // NW
    n_chunks = n_per_w // CHUNK
    mesh = plsc.VectorSubcoreMesh(core_axis_name="c", subcore_axis_name="s")

    @functools.partial(
        pl.kernel,
        mesh=mesh,
        out_type=jax.ShapeDtypeStruct((n_rows, d), jnp.float32),
        scratch_types=[
            pltpu.VMEM((K, SUB), jnp.int32),
            pltpu.VMEM((CHUNK, d), jnp.float32),
            pltpu.SemaphoreType.DMA,
        ],
    )
    def grab(idx_hbm, table_hbm, out_hbm, idx_v, rows_v, sem):
        wid = lax.axis_index("s") * NC + lax.axis_index("c")
        row0 = wid * n_per_w

        def body(i, carry):
            off = row0 + i * CHUNK
            pltpu.sync_copy(idx_hbm.at[pl.ds(off // SUB, K)], idx_v)
            descs = [
                pltpu.async_copy(
                    table_hbm.at[idx_v.at[j]],
                    rows_v.at[pl.ds(j * SUB, SUB)],
                    sem,
                )
                for j in range(K)
            ]
            for cp in descs:
                cp.wait()
            pltpu.sync_copy(rows_v, out_hbm.at[pl.ds(off, CHUNK)])
            return carry

        lax.fori_loop(0, n_chunks, body, 0)

    return grab


def kernel(ids, table):
    b, l = ids.shape
    v, d = table.shape
    n = b * l
    flat = jnp.reshape(ids.astype(jnp.int32), (n // SUB, SUB))
    out = _gather_call(n, v, d)(flat, table)
    return jnp.reshape(out, (b, l, d))


# speedup vs baseline: 5.4706x; 5.4706x over previous
import functools

import jax
import jax.numpy as jnp
from jax import lax
from jax.experimental import pallas as pl
from jax.experimental.pallas import tpu as pltpu
from jax.experimental.pallas import tpu_sc as plsc

NC = 2
NS = 16
NW = NC * NS

SUB = 128
K = 8
CHUNK = SUB * K


def _gather_call(n_rows, vocab, d):
    n_per_w = n_rows // NW
    n_chunks = n_per_w // CHUNK
    mesh = plsc.VectorSubcoreMesh(core_axis_name="c", subcore_axis_name="s")

    @functools.partial(
        pl.kernel,
        mesh=mesh,
        compiler_params=pltpu.CompilerParams(use_tc_tiling_on_sc=False),
        out_type=jax.ShapeDtypeStruct((n_rows, d), jnp.float32),
        scratch_types=[
            pltpu.VMEM((K, SUB), jnp.int32),
            pltpu.VMEM((CHUNK, d), jnp.float32),
            pltpu.SemaphoreType.DMA,
        ],
    )
    def grab(idx_hbm, table_hbm, out_hbm, idx_v, rows_v, sem):
        wid = lax.axis_index("s") * NC + lax.axis_index("c")
        row0 = wid * n_per_w

        def body(i, carry):
            off = pl.multiple_of(row0 + i * CHUNK, CHUNK)
            pltpu.sync_copy(idx_hbm.at[pl.ds(pl.multiple_of(off // SUB, K), K)], idx_v)
            descs = [
                pltpu.async_copy(
                    table_hbm.at[idx_v.at[j]],
                    rows_v.at[pl.ds(j * SUB, SUB)],
                    sem,
                )
                for j in range(K)
            ]
            for cp in descs:
                cp.wait()
            pltpu.sync_copy(rows_v, out_hbm.at[pl.ds(off, CHUNK)])
            return carry

        lax.fori_loop(0, n_chunks, body, 0)

    return grab


def kernel(ids, table):
    b, l = ids.shape
    v, d = table.shape
    n = b * l
    flat = jnp.reshape(ids.astype(jnp.int32), (n // SUB, SUB))
    out = _gather_call(n, v, d)(flat, table)
    return jnp.reshape(out, (b, l, d))
